# baseline (speedup 1.0000x reference)
import functools

import jax
import jax.numpy as jnp
from jax import lax
from jax.experimental import pallas as pl
from jax.experimental.pallas import tpu as pltpu
from jax.experimental.pallas import tpu_sc as plsc

_N = 50000
_E = 1600000
_NIN = 24
_NPLANE = 24
_GROUPS = _E // 128
_PLANE_STRIDE = _E

_NC = 2
_NS = 16
_NW = _NC * _NS

_G = 5
_CE = _G * 128
_CSEG = _G * 128
_CIDX = _G * 128
_CVREG = _CE // 16
_NCHUNK = _GROUPS // _G
_ITER2 = (-(-_NCHUNK // _NW) + 1) // 2

_NB = 1000


def _mlp_body(x_ref, w1_ref, b1_ref, w2_ref, b2_ref, o_ref):
    z = jnp.dot(x_ref[...], w1_ref[...], preferred_element_type=jnp.float32)
    z = z + b1_ref[...]
    h = z * (1.0 / (1.0 + jnp.exp(-z)))
    o_ref[...] = jnp.dot(h, w2_ref[...], preferred_element_type=jnp.float32) + b2_ref[...]


def _node_mlp(x2d, W1, b1, W2, b2):
    grid = (_N // _NB,)
    return pl.pallas_call(
        _mlp_body,
        grid=grid,
        in_specs=[
            pl.BlockSpec((_NB, _NIN), lambda i: (i, 0)),
            pl.BlockSpec((_NIN, 128), lambda i: (0, 0)),
            pl.BlockSpec((1, 128), lambda i: (0, 0)),
            pl.BlockSpec((128, 1), lambda i: (0, 0)),
            pl.BlockSpec((1, 1), lambda i: (0, 0)),
        ],
        out_specs=pl.BlockSpec((_NB, 1), lambda i: (i, 0)),
        out_shape=jax.ShapeDtypeStruct((_N, 1), jnp.float32),
    )(x2d, W1, b1.reshape(1, 128), W2, b2.reshape(1, 1))


def _fused_body(nw_hbm, idx_hbm, attr_hbm, out_hbm,
                table_v, idx_v0, idx_v1, ew_v0, ew_v1,
                attr_v0, attr_v1, prod_v0, prod_v1,
                sin0, sin1, sout0, sout1):
    idx_v = (idx_v0, idx_v1)
    ew_v = (ew_v0, ew_v1)
    attr_v = (attr_v0, attr_v1)
    prod_v = (prod_v0, prod_v1)
    sin = (sin0, sin1)
    sout = (sout0, sout1)

    wid = lax.axis_index("s") * _NC + lax.axis_index("c")
    pltpu.sync_copy(nw_hbm, table_v)

    def start_in(c, b):
        pltpu.async_copy(idx_hbm.at[pl.ds(c * _CIDX, _CIDX)], idx_v[b], sin[b])
        for p in range(_NPLANE):
            pltpu.async_copy(
                attr_hbm.at[pl.ds(p * _PLANE_STRIDE + c * _CSEG, _CSEG)],
                attr_v[b].at[pl.ds(p * _CSEG, _CSEG)],
                sin[b],
            )

    def wait_in(b):
        pltpu.make_async_copy(idx_hbm.at[pl.ds(0, _CIDX)], idx_v[b], sin[b]).wait()
        pltpu.make_async_copy(
            attr_hbm.at[pl.ds(0, _NPLANE * _CSEG)], attr_v[b], sin[b]
        ).wait()

    def start_out(c, b):
        for p in range(_NPLANE):
            pltpu.async_copy(
                prod_v[b].at[pl.ds(p * _CSEG, _CSEG)],
                out_hbm.at[pl.ds(p * _PLANE_STRIDE + c * _CSEG, _CSEG)],
                sout[b],
            )

    def wait_out(b):
        pltpu.make_async_copy(
            prod_v[b], out_hbm.at[pl.ds(0, _NPLANE * _CSEG)], sout[b]
        ).wait()

    start_in(wid, 0)

    def outer(i2, carry):
        for b in range(2):
            c = wid + (2 * i2 + b) * _NW
            cp = c - 2 * _NW
            cn = c + _NW

            @pl.when(cn < _NCHUNK)
            def _():
                start_in(cn, 1 - b)

            @pl.when(cp >= 0)
            def _():
                wait_out(b)

            @pl.when(c < _NCHUNK)
            def _():
                wait_in(b)

                @plsc.parallel_loop(0, _CVREG, unroll=4)
                def gather_w(st):
                    iv = idx_v[b][pl.ds(st * 16, 16)]
                    ew_v[b][pl.ds(st * 16, 16)] = plsc.load_gather(table_v, [iv])

                @plsc.parallel_loop(0, _CVREG, unroll=2)
                def mul_all(st):
                    m = ew_v[b][pl.ds(st * 16, 16)]
                    base = st * 16
                    for p in range(_NPLANE):
                        a = p * _CSEG + base
                        prod_v[b][pl.ds(a, 16)] = attr_v[b][pl.ds(a, 16)] * m

                start_out(c, b)

        return carry

    lax.fori_loop(0, _ITER2, outer, 0)

    for b in range(2):
        c_last = wid + (2 * (_ITER2 - 1) + b) * _NW

        @pl.when(c_last < _NCHUNK)
        def _():
            wait_out(b)


def _edge_fused(nw_flat, idx_flat, attr_flat):
    mesh = plsc.VectorSubcoreMesh(core_axis_name="c", subcore_axis_name="s")
    call = pl.kernel(
        _fused_body,
        out_type=jax.ShapeDtypeStruct((_E * _NIN,), jnp.float32),
        mesh=mesh,
        scratch_types=[
            pltpu.VMEM((_N,), jnp.float32),
            pltpu.VMEM((_CIDX,), jnp.int32),
            pltpu.VMEM((_CIDX,), jnp.int32),
            pltpu.VMEM((_CE,), jnp.float32),
            pltpu.VMEM((_CE,), jnp.float32),
            pltpu.VMEM((_NPLANE * _CSEG,), jnp.float32),
            pltpu.VMEM((_NPLANE * _CSEG,), jnp.float32),
            pltpu.VMEM((_NPLANE * _CSEG,), jnp.float32),
            pltpu.VMEM((_NPLANE * _CSEG,), jnp.float32),
            pltpu.SemaphoreType.DMA,
            pltpu.SemaphoreType.DMA,
            pltpu.SemaphoreType.DMA,
            pltpu.SemaphoreType.DMA,
        ],
        compiler_params=pltpu.CompilerParams(needs_layout_passes=False),
    )
    return call(nw_flat, idx_flat, attr_flat)


def _to_native_flat(a4d):
    return a4d.transpose(1, 2, 3, 0).reshape(_E * _NIN)


def _from_native_flat(flat):
    t = flat.reshape(4, 3, 2, _E)
    return t.transpose(3, 0, 1, 2)


def _idx_native_flat(edge_index):
    return edge_index.reshape(2 * _E)


def kernel(node_feat, edge_attri, edge_index, W1, b1, W2, b2):
    x2d = node_feat.reshape(_N, _NIN)
    nw = _node_mlp(x2d, W1, b1, W2, b2)
    out_flat = _edge_fused(
        nw.reshape(_N),
        _idx_native_flat(edge_index),
        _to_native_flat(edge_attri),
    )
    return _from_native_flat(out_flat)


# speedup vs baseline: 1.2290x; 1.2290x over previous
import functools

import jax
import jax.numpy as jnp
from jax import lax
from jax.experimental import pallas as pl
from jax.experimental.pallas import tpu as pltpu
from jax.experimental.pallas import tpu_sc as plsc

_N = 50000
_E = 1600000
_NIN = 24
_NPLANE = 12
_GROUPS = _E // 128
_PLANE_STRIDE = _GROUPS * 256

_NC = 2
_NS = 16
_NW = _NC * _NS

_G = 5
_CE = _G * 128
_CSEG = _G * 256
_CIDX = _G * 256
_CVREG = _CE // 16
_NCHUNK = _GROUPS // _G
_ITER2 = (-(-_NCHUNK // _NW) + 1) // 2

_NB = 1000


def _mlp_body(x_ref, w1_ref, b1_ref, w2_ref, b2_ref, o_ref):
    z = jnp.dot(x_ref[...], w1_ref[...], preferred_element_type=jnp.float32)
    z = z + b1_ref[...]
    h = z * (1.0 / (1.0 + jnp.exp(-z)))
    o_ref[...] = jnp.dot(h, w2_ref[...], preferred_element_type=jnp.float32) + b2_ref[...]


def _node_mlp(x2d, W1, b1, W2, b2):
    grid = (_N // _NB,)
    return pl.pallas_call(
        _mlp_body,
        grid=grid,
        in_specs=[
            pl.BlockSpec((_NB, _NIN), lambda i: (i, 0)),
            pl.BlockSpec((_NIN, 128), lambda i: (0, 0)),
            pl.BlockSpec((1, 128), lambda i: (0, 0)),
            pl.BlockSpec((128, 1), lambda i: (0, 0)),
            pl.BlockSpec((1, 1), lambda i: (0, 0)),
        ],
        out_specs=pl.BlockSpec((_NB, 1), lambda i: (i, 0)),
        out_shape=jax.ShapeDtypeStruct((_N, 1), jnp.float32),
    )(x2d, W1, b1.reshape(1, 128), W2, b2.reshape(1, 1))


def _fused_body(nw_hbm, idx_hbm, attr_hbm, out_hbm,
                table_v, idx_v0, idx_v1, ew_v0, ew_v1,
                attr_v0, attr_v1, prod_v0, prod_v1,
                sin0, sin1, sout0, sout1):
    idx_v = (idx_v0, idx_v1)
    ew_v = (ew_v0, ew_v1)
    attr_v = (attr_v0, attr_v1)
    prod_v = (prod_v0, prod_v1)
    sin = (sin0, sin1)
    sout = (sout0, sout1)

    wid = lax.axis_index("s") * _NC + lax.axis_index("c")
    pltpu.sync_copy(nw_hbm, table_v)

    def start_in(c, b):
        pltpu.async_copy(idx_hbm.at[pl.ds(c * _CIDX, _CIDX)], idx_v[b], sin[b])
        for p in range(_NPLANE):
            pltpu.async_copy(
                attr_hbm.at[pl.ds(p * _PLANE_STRIDE + c * _CSEG, _CSEG)],
                attr_v[b].at[pl.ds(p * _CSEG, _CSEG)],
                sin[b],
            )

    def wait_in(b):
        pltpu.make_async_copy(idx_hbm.at[pl.ds(0, _CIDX)], idx_v[b], sin[b]).wait()
        pltpu.make_async_copy(
            attr_hbm.at[pl.ds(0, _NPLANE * _CSEG)], attr_v[b], sin[b]
        ).wait()

    def start_out(c, b):
        for p in range(_NPLANE):
            pltpu.async_copy(
                prod_v[b].at[pl.ds(p * _CSEG, _CSEG)],
                out_hbm.at[pl.ds(p * _PLANE_STRIDE + c * _CSEG, _CSEG)],
                sout[b],
            )

    def wait_out(b):
        pltpu.make_async_copy(
            prod_v[b], out_hbm.at[pl.ds(0, _NPLANE * _CSEG)], sout[b]
        ).wait()

    start_in(wid, 0)

    def outer(i2, carry):
        for b in range(2):
            c = wid + (2 * i2 + b) * _NW
            cp = c - 2 * _NW
            cn = c + _NW

            @pl.when(cn < _NCHUNK)
            def _():
                start_in(cn, 1 - b)

            @pl.when(cp >= 0)
            def _():
                wait_out(b)

            @pl.when(c < _NCHUNK)
            def _():
                wait_in(b)

                @plsc.parallel_loop(0, _CVREG, unroll=4)
                def gather_w(st):
                    g = st // 8
                    s = st % 8
                    iv = idx_v[b][pl.ds(g * 256 + s * 16, 16)]
                    ew_v[b][pl.ds(st * 16, 16)] = plsc.load_gather(table_v, [iv])

                @plsc.parallel_loop(0, _CVREG, unroll=2)
                def mul_all(st):
                    m = ew_v[b][pl.ds(st * 16, 16)]
                    base = (st // 8) * 256 + (st % 8) * 16
                    for p in range(_NPLANE):
                        for d3 in range(2):
                            a = p * _CSEG + base + d3 * 128
                            prod_v[b][pl.ds(a, 16)] = attr_v[b][pl.ds(a, 16)] * m

                start_out(c, b)

        return carry

    lax.fori_loop(0, _ITER2, outer, 0)

    for b in range(2):
        c_last = wid + (2 * (_ITER2 - 1) + b) * _NW

        @pl.when(c_last < _NCHUNK)
        def _():
            wait_out(b)


def _edge_fused(nw_flat, idx_flat, attr_flat):
    mesh = plsc.VectorSubcoreMesh(core_axis_name="c", subcore_axis_name="s")
    call = pl.kernel(
        _fused_body,
        out_type=jax.ShapeDtypeStruct((_E * _NIN,), jnp.float32),
        mesh=mesh,
        scratch_types=[
            pltpu.VMEM((_N,), jnp.float32),
            pltpu.VMEM((_CIDX,), jnp.int32),
            pltpu.VMEM((_CIDX,), jnp.int32),
            pltpu.VMEM((_CE,), jnp.float32),
            pltpu.VMEM((_CE,), jnp.float32),
            pltpu.VMEM((_NPLANE * _CSEG,), jnp.float32),
            pltpu.VMEM((_NPLANE * _CSEG,), jnp.float32),
            pltpu.VMEM((_NPLANE * _CSEG,), jnp.float32),
            pltpu.VMEM((_NPLANE * _CSEG,), jnp.float32),
            pltpu.SemaphoreType.DMA,
            pltpu.SemaphoreType.DMA,
            pltpu.SemaphoreType.DMA,
            pltpu.SemaphoreType.DMA,
        ],
        compiler_params=pltpu.CompilerParams(needs_layout_passes=False),
    )
    return call(nw_flat, idx_flat, attr_flat)


def _to_native_flat(a4d):
    r = a4d.reshape(_GROUPS, 128, 4, 3, 2)
    p = r.transpose(2, 3, 0, 4, 1)
    return p.reshape(_E * _NIN)


def _from_native_flat(flat):
    p = flat.reshape(4, 3, _GROUPS, 2, 128)
    r = p.transpose(0, 1, 3, 2, 4)
    t = r.reshape(4, 3, 2, _E)
    return t.transpose(3, 0, 1, 2)


def _idx_native_flat(edge_index):
    return edge_index.reshape(2, _GROUPS, 128).transpose(1, 0, 2).reshape(2 * _E)


def kernel(node_feat, edge_attri, edge_index, W1, b1, W2, b2):
    x2d = node_feat.reshape(_N, _NIN)
    nw = _node_mlp(x2d, W1, b1, W2, b2)
    out_flat = _edge_fused(
        nw.reshape(_N),
        _idx_native_flat(edge_index),
        _to_native_flat(edge_attri),
    )
    return _from_native_flat(out_flat)
